# Initial kernel scaffold; baseline (speedup 1.0000x reference)
import jax
import jax.numpy as jnp
from jax import lax
from jax.experimental import pallas as pl
from jax.experimental.pallas import tpu as pltpu
from jax.experimental.pallas import tpu_sc as plsc

_N = 10000
_E = 320000
_D = 128
_G = 64
_BN_EPS = 1e-5

_NC = 2
_NS = 16
_K = 80
_EPT = _E // (_NC * _NS)
_T = _EPT // _K
_RPT = _N // _NS

_mesh = plsc.VectorSubcoreMesh(core_axis_name="c", subcore_axis_name="s")


def _agg_body(x_hbm, src_hbm, dst_hbm, zero_hbm, out_hbm,
              is0, is1, id0, id1, rows0, rows1, acc,
              sis0, sis1, sid0, sid1, sg0, sg1):
    c = lax.axis_index("c")
    s = lax.axis_index("s")

    row0 = s * _RPT

    @pl.when(c == 0)
    def _():
        pltpu.sync_copy(x_hbm.at[pl.ds(row0, _RPT)], acc.at[pl.ds(row0, _RPT)])

    @pl.when(c != 0)
    def _():
        pltpu.sync_copy(zero_hbm.at[pl.ds(row0, _RPT)],
                        acc.at[pl.ds(row0, _RPT)])

    plsc.subcore_barrier()

    ebase = c * (_E // _NC) + s * _EPT
    last = _T - 1

    idx_s = (is0, is1)
    idx_d = (id0, id1)
    rows = (rows0, rows1)
    sem_is = (sis0, sis1)
    sem_id = (sid0, sid1)
    sem_g = (sg0, sg1)

    def start_idx(chunk, b):
        off = ebase + chunk * _K
        pltpu.async_copy(src_hbm.at[pl.ds(off, _K)], idx_s[b], sem_is[b])
        pltpu.async_copy(dst_hbm.at[pl.ds(off, _K)], idx_d[b], sem_id[b])

    def wait_idx(b):
        pltpu.make_async_copy(src_hbm.at[pl.ds(0, _K)], idx_s[b],
                              sem_is[b]).wait()
        pltpu.make_async_copy(dst_hbm.at[pl.ds(0, _K)], idx_d[b],
                              sem_id[b]).wait()

    def start_gather(b):
        pltpu.async_copy(x_hbm.at[idx_s[b]], rows[b], sem_g[b])

    def wait_gather(b):
        pltpu.make_async_copy(x_hbm.at[idx_s[b]], rows[b], sem_g[b]).wait()

    def step(i, p, q):
        wait_idx(q)
        start_gather(q)
        wait_gather(p)
        start_idx(jnp.minimum(i + 2, last), p)
        pltpu.sync_copy(rows[p], acc.at[idx_d[p]], add=True)

    start_idx(0, 0)
    wait_idx(0)
    start_gather(0)
    start_idx(1, 1)

    def loop_body(j, carry):
        i = j * 2
        step(i, 0, 1)
        step(i + 1, 1, 0)
        return carry

    lax.fori_loop(0, (_T - 1) // 2, loop_body, 0)
    wait_idx(0)
    start_gather(0)
    wait_gather(0)
    pltpu.sync_copy(rows[0], acc.at[idx_d[0]], add=True)

    plsc.subcore_barrier()
    pltpu.sync_copy(acc.at[pl.ds(row0, _RPT)],
                    out_hbm.at[c, pl.ds(row0, _RPT)])


_agg = pl.kernel(
    _agg_body,
    out_type=jax.ShapeDtypeStruct((_NC, _N, _D), jnp.float32),
    mesh=_mesh,
    scratch_types=[
        pltpu.VMEM((_K,), jnp.int32),
        pltpu.VMEM((_K,), jnp.int32),
        pltpu.VMEM((_K,), jnp.int32),
        pltpu.VMEM((_K,), jnp.int32),
        pltpu.VMEM((_K, _D), jnp.float32),
        pltpu.VMEM((_K, _D), jnp.float32),
        pltpu.VMEM_SHARED((_N, _D), jnp.float32),
        pltpu.SemaphoreType.DMA,
        pltpu.SemaphoreType.DMA,
        pltpu.SemaphoreType.DMA,
        pltpu.SemaphoreType.DMA,
        pltpu.SemaphoreType.DMA,
        pltpu.SemaphoreType.DMA,
    ],
)


def _pool_body(h_hbm, batch_hbm, zero_hbm, out_hbm, bidx, rows, acc):
    c = lax.axis_index("c")
    s = lax.axis_index("s")
    w = c * _NS + s

    @pl.when(s == 0)
    def _():
        pltpu.sync_copy(zero_hbm.at[pl.ds(0, _G)], acc)

    plsc.subcore_barrier()

    nchunks = _N // _K

    def body(j, carry):
        chunk = w + j * (_NC * _NS)

        @pl.when(chunk < nchunks)
        def _():
            off = chunk * _K
            pltpu.sync_copy(batch_hbm.at[pl.ds(off, _K)], bidx)
            pltpu.sync_copy(h_hbm.at[pl.ds(off, _K)], rows)
            pltpu.sync_copy(rows, acc.at[bidx], add=True)

        return carry

    lax.fori_loop(0, (nchunks + _NC * _NS - 1) // (_NC * _NS), body, 0)
    plsc.subcore_barrier()

    @pl.when(s == 0)
    def _():
        pltpu.sync_copy(acc, out_hbm.at[c])


_pool = pl.kernel(
    _pool_body,
    out_type=jax.ShapeDtypeStruct((_NC, _G, _D), jnp.float32),
    mesh=_mesh,
    scratch_types=[
        pltpu.VMEM((_K,), jnp.int32),
        pltpu.VMEM((_K, _D), jnp.float32),
        pltpu.VMEM_SHARED((_G, _D), jnp.float32),
    ],
)

_BLK = 400


def _mlp_body(a_ref, w1_ref, b1_ref, w2_ref, b2_ref, out_ref):
    s = a_ref[0] + a_ref[1]
    h = jnp.dot(s, w1_ref[...], preferred_element_type=jnp.float32,
                precision=lax.Precision.HIGHEST)
    h = jnp.maximum(h + b1_ref[...], 0.0)
    o = jnp.dot(h, w2_ref[...], preferred_element_type=jnp.float32,
                precision=lax.Precision.HIGHEST)
    out_ref[...] = jnp.maximum(o + b2_ref[...], 0.0)


def _mlp(a, w1, b1, w2, b2):
    return pl.pallas_call(
        _mlp_body,
        grid=(_N // _BLK,),
        in_specs=[
            pl.BlockSpec((_NC, _BLK, _D), lambda i: (0, i, 0)),
            pl.BlockSpec((_D, _D), lambda i: (0, 0)),
            pl.BlockSpec((1, _D), lambda i: (0, 0)),
            pl.BlockSpec((_D, _D), lambda i: (0, 0)),
            pl.BlockSpec((1, _D), lambda i: (0, 0)),
        ],
        out_specs=pl.BlockSpec((_BLK, _D), lambda i: (i, 0)),
        out_shape=jax.ShapeDtypeStruct((_N, _D), jnp.float32),
    )(a, w1, b1, w2, b2)


def _final_body(p_ref, w_ref, b_ref, out_ref):
    pooled = p_ref[0] + p_ref[1]
    z = jnp.dot(pooled, w_ref[...], preferred_element_type=jnp.float32,
                precision=lax.Precision.HIGHEST)
    out_ref[...] = z + b_ref[...]


def _final(p, lin_W, lin_b):
    return pl.pallas_call(
        _final_body,
        out_shape=jax.ShapeDtypeStruct((_G, 1), jnp.float32),
    )(p, lin_W, lin_b.reshape(1, 1))


def kernel(x, edge_index, batch,
           c1_W1, c1_b1, c1_g, c1_be, c1_W2, c1_b2,
           c2_W1, c2_b1, c2_g, c2_be, c2_W2, c2_b2,
           c3_W1, c3_b1, c3_g, c3_be, c3_W2, c3_b2,
           lin_W, lin_b):
    src = edge_index[0]
    dst = edge_index[1]
    zeros = jnp.zeros((_N, _D), jnp.float32)

    h = x
    for (W1, b1, g, be, W2, b2) in (
            (c1_W1, c1_b1, c1_g, c1_be, c1_W2, c1_b2),
            (c2_W1, c2_b1, c2_g, c2_be, c2_W2, c2_b2),
            (c3_W1, c3_b1, c3_g, c3_be, c3_W2, c3_b2)):
        scale = g / jnp.sqrt(1.0 + _BN_EPS)
        w1f = W1 * scale[None, :]
        b1f = (b1 * scale + be)[None, :]
        a = _agg(h, src, dst, zeros)
        h = _mlp(a, w1f, b1f, W2, b2[None, :])

    p = _pool(h, batch, zeros)
    return _final(p, lin_W, lin_b)


# speedup vs baseline: 10.2611x; 10.2611x over previous
import jax
import jax.numpy as jnp
from jax import lax
from jax.experimental import pallas as pl
from jax.experimental.pallas import tpu as pltpu
from jax.experimental.pallas import tpu_sc as plsc

_N = 10000
_E = 320000
_D = 128
_G = 64
_BN_EPS = 1e-5

_NC = 2
_NS = 16
_K = 80
_EPT = _E // (_NC * _NS)
_T = _EPT // _K
_RPT_BIG = 632
_RPT_LAST = _N - 15 * _RPT_BIG

_mesh = plsc.VectorSubcoreMesh(core_axis_name="c", subcore_axis_name="s")


def _agg_body(x_hbm, src_hbm, dst_hbm, zero_hbm, out_hbm,
              is0, is1, id0, id1, rows0, rows1, acc,
              sis0, sis1, sid0, sid1, sg0, sg1):
    c = lax.axis_index("c")
    s = lax.axis_index("s")

    row0 = s * _RPT_BIG

    def init_from(src_hbm_ref):
        @pl.when(s < _NS - 1)
        def _():
            pltpu.sync_copy(src_hbm_ref.at[pl.ds(row0, _RPT_BIG)],
                            acc.at[pl.ds(row0, _RPT_BIG)])

        @pl.when(s == _NS - 1)
        def _():
            pltpu.sync_copy(src_hbm_ref.at[pl.ds(row0, _RPT_LAST)],
                            acc.at[pl.ds(row0, _RPT_LAST)])

    @pl.when(c == 0)
    def _():
        init_from(x_hbm)

    @pl.when(c != 0)
    def _():
        init_from(zero_hbm)

    plsc.subcore_barrier()

    ebase = c * (_E // _NC) + s * _EPT
    last = _T - 1

    idx_s = (is0, is1)
    idx_d = (id0, id1)
    rows = (rows0, rows1)
    sem_is = (sis0, sis1)
    sem_id = (sid0, sid1)
    sem_g = (sg0, sg1)

    def start_idx(chunk, b):
        off = ebase + chunk * _K
        pltpu.async_copy(src_hbm.at[pl.ds(off, _K)], idx_s[b], sem_is[b])
        pltpu.async_copy(dst_hbm.at[pl.ds(off, _K)], idx_d[b], sem_id[b])

    def wait_idx(b):
        pltpu.make_async_copy(src_hbm.at[pl.ds(0, _K)], idx_s[b],
                              sem_is[b]).wait()
        pltpu.make_async_copy(dst_hbm.at[pl.ds(0, _K)], idx_d[b],
                              sem_id[b]).wait()

    def start_gather(b):
        pltpu.async_copy(x_hbm.at[idx_s[b]], rows[b], sem_g[b])

    def wait_gather(b):
        pltpu.make_async_copy(x_hbm.at[idx_s[b]], rows[b], sem_g[b]).wait()

    def step(i, p, q):
        wait_idx(q)
        start_gather(q)
        wait_gather(p)

        @pl.when(i + 2 <= last)
        def _():
            start_idx(i + 2, p)

        pltpu.sync_copy(rows[p], acc.at[idx_d[p]], add=True)

    start_idx(0, 0)
    wait_idx(0)
    start_gather(0)
    start_idx(1, 1)

    def loop_body(j, carry):
        i = j * 2
        step(i, 0, 1)
        step(i + 1, 1, 0)
        return carry

    lax.fori_loop(0, (_T - 1) // 2, loop_body, 0)
    wait_gather(0)
    pltpu.sync_copy(rows[0], acc.at[idx_d[0]], add=True)

    plsc.subcore_barrier()

    @pl.when(s < _NS - 1)
    def _():
        pltpu.sync_copy(acc.at[pl.ds(row0, _RPT_BIG)],
                        out_hbm.at[c, pl.ds(row0, _RPT_BIG)])

    @pl.when(s == _NS - 1)
    def _():
        pltpu.sync_copy(acc.at[pl.ds(row0, _RPT_LAST)],
                        out_hbm.at[c, pl.ds(row0, _RPT_LAST)])


_agg = pl.kernel(
    _agg_body,
    out_type=jax.ShapeDtypeStruct((_NC, _N, _D), jnp.float32),
    mesh=_mesh,
    scratch_types=[
        pltpu.VMEM((_K,), jnp.int32),
        pltpu.VMEM((_K,), jnp.int32),
        pltpu.VMEM((_K,), jnp.int32),
        pltpu.VMEM((_K,), jnp.int32),
        pltpu.VMEM((_K, _D), jnp.float32),
        pltpu.VMEM((_K, _D), jnp.float32),
        pltpu.VMEM_SHARED((_N, _D), jnp.float32),
        pltpu.SemaphoreType.DMA,
        pltpu.SemaphoreType.DMA,
        pltpu.SemaphoreType.DMA,
        pltpu.SemaphoreType.DMA,
        pltpu.SemaphoreType.DMA,
        pltpu.SemaphoreType.DMA,
    ],
)


def _pool_body(h_hbm, batch_hbm, zero_hbm, out_hbm, bidx, rows, acc):
    c = lax.axis_index("c")
    s = lax.axis_index("s")
    w = c * _NS + s

    @pl.when(s == 0)
    def _():
        pltpu.sync_copy(zero_hbm.at[pl.ds(0, _G)], acc)

    plsc.subcore_barrier()

    nchunks = _N // _K

    def body(j, carry):
        chunk = w + j * (_NC * _NS)

        @pl.when(chunk < nchunks)
        def _():
            off = chunk * _K
            pltpu.sync_copy(batch_hbm.at[pl.ds(off, _K)], bidx)
            pltpu.sync_copy(h_hbm.at[pl.ds(off, _K)], rows)
            pltpu.sync_copy(rows, acc.at[bidx], add=True)

        return carry

    lax.fori_loop(0, (nchunks + _NC * _NS - 1) // (_NC * _NS), body, 0)
    plsc.subcore_barrier()

    @pl.when(s == 0)
    def _():
        pltpu.sync_copy(acc, out_hbm.at[c])


_pool = pl.kernel(
    _pool_body,
    out_type=jax.ShapeDtypeStruct((_NC, _G, _D), jnp.float32),
    mesh=_mesh,
    scratch_types=[
        pltpu.VMEM((_K,), jnp.int32),
        pltpu.VMEM((_K, _D), jnp.float32),
        pltpu.VMEM_SHARED((_G, _D), jnp.float32),
    ],
)

_BLK = 400


def _mlp_body(a_ref, w1_ref, b1_ref, w2_ref, b2_ref, out_ref):
    s = a_ref[0] + a_ref[1]
    h = jnp.dot(s, w1_ref[...], preferred_element_type=jnp.float32,
                precision=lax.Precision.HIGHEST)
    h = jnp.maximum(h + b1_ref[...], 0.0)
    o = jnp.dot(h, w2_ref[...], preferred_element_type=jnp.float32,
                precision=lax.Precision.HIGHEST)
    out_ref[...] = jnp.maximum(o + b2_ref[...], 0.0)


def _mlp(a, w1, b1, w2, b2):
    return pl.pallas_call(
        _mlp_body,
        grid=(_N // _BLK,),
        in_specs=[
            pl.BlockSpec((_NC, _BLK, _D), lambda i: (0, i, 0)),
            pl.BlockSpec((_D, _D), lambda i: (0, 0)),
            pl.BlockSpec((1, _D), lambda i: (0, 0)),
            pl.BlockSpec((_D, _D), lambda i: (0, 0)),
            pl.BlockSpec((1, _D), lambda i: (0, 0)),
        ],
        out_specs=pl.BlockSpec((_BLK, _D), lambda i: (i, 0)),
        out_shape=jax.ShapeDtypeStruct((_N, _D), jnp.float32),
    )(a, w1, b1, w2, b2)


def _final_body(p_ref, w_ref, b_ref, out_ref):
    pooled = p_ref[0] + p_ref[1]
    z = jnp.dot(pooled, w_ref[...], preferred_element_type=jnp.float32,
                precision=lax.Precision.HIGHEST)
    out_ref[...] = z + b_ref[...]


def _final(p, lin_W, lin_b):
    return pl.pallas_call(
        _final_body,
        out_shape=jax.ShapeDtypeStruct((_G, 1), jnp.float32),
    )(p, lin_W, lin_b.reshape(1, 1))


def kernel(x, edge_index, batch,
           c1_W1, c1_b1, c1_g, c1_be, c1_W2, c1_b2,
           c2_W1, c2_b1, c2_g, c2_be, c2_W2, c2_b2,
           c3_W1, c3_b1, c3_g, c3_be, c3_W2, c3_b2,
           lin_W, lin_b):
    src = edge_index[0]
    dst = edge_index[1]
    zeros = jnp.zeros((_N, _D), jnp.float32)

    h = x
    for (W1, b1, g, be, W2, b2) in (
            (c1_W1, c1_b1, c1_g, c1_be, c1_W2, c1_b2),
            (c2_W1, c2_b1, c2_g, c2_be, c2_W2, c2_b2),
            (c3_W1, c3_b1, c3_g, c3_be, c3_W2, c3_b2)):
        scale = g / jnp.sqrt(1.0 + _BN_EPS)
        w1f = W1 * scale[None, :]
        b1f = (b1 * scale + be)[None, :]
        a = _agg(h, src, dst, zeros)
        h = _mlp(a, w1f, b1f, W2, b2[None, :])

    p = _pool(h, batch, zeros)
    return _final(p, lin_W, lin_b)


# speedup vs baseline: 10.4783x; 1.0212x over previous
import jax
import jax.numpy as jnp
from jax import lax
from jax.experimental import pallas as pl
from jax.experimental.pallas import tpu as pltpu
from jax.experimental.pallas import tpu_sc as plsc

_N = 10000
_E = 320000
_D = 128
_G = 64
_BN_EPS = 1e-5

_NC = 2
_NS = 16
_K = 80
_EPT = _E // (_NC * _NS)
_T = _EPT // _K
_RPT_BIG = 632
_RPT_LAST = _N - 15 * _RPT_BIG

_mesh = plsc.VectorSubcoreMesh(core_axis_name="c", subcore_axis_name="s")


def _agg_body(x_hbm, src_hbm, dst_hbm, zero_hbm, out_hbm,
              srcall, dstall, rows0, rows1, acc,
              sidx, sg0, sg1, ss0, ss1):
    c = lax.axis_index("c")
    s = lax.axis_index("s")

    row0 = s * _RPT_BIG

    def init_from(src_hbm_ref):
        @pl.when(s < _NS - 1)
        def _():
            pltpu.sync_copy(src_hbm_ref.at[pl.ds(row0, _RPT_BIG)],
                            acc.at[pl.ds(row0, _RPT_BIG)])

        @pl.when(s == _NS - 1)
        def _():
            pltpu.sync_copy(src_hbm_ref.at[pl.ds(row0, _RPT_LAST)],
                            acc.at[pl.ds(row0, _RPT_LAST)])

    @pl.when(c == 0)
    def _():
        init_from(x_hbm)

    @pl.when(c != 0)
    def _():
        init_from(zero_hbm)

    pltpu.async_copy(src_hbm.at[c, s], srcall, sidx)
    pltpu.async_copy(dst_hbm.at[c, s], dstall, sidx)
    pltpu.make_async_copy(src_hbm.at[c, s], srcall, sidx).wait()
    pltpu.make_async_copy(dst_hbm.at[c, s], dstall, sidx).wait()

    plsc.subcore_barrier()

    rows = (rows0, rows1)
    sem_g = (sg0, sg1)
    sem_s = (ss0, ss1)

    def start_gather(i, b):
        pltpu.async_copy(x_hbm.at[srcall.at[pl.ds(i * _K, _K)]], rows[b],
                         sem_g[b])

    def wait_gather(i, b):
        pltpu.make_async_copy(x_hbm.at[srcall.at[pl.ds(i * _K, _K)]],
                              rows[b], sem_g[b]).wait()

    def start_scatter(i, b):
        pltpu.async_copy(rows[b], acc.at[dstall.at[i]], sem_s[b], add=True)

    def wait_scatter(i, b):
        pltpu.make_async_copy(rows[b], acc.at[dstall.at[i]],
                              sem_s[b]).wait()

    def mk_step(i, bp, bq, with_wait, with_gather):
        if with_wait:
            wait_scatter(i - 1, bq)
        if with_gather:
            start_gather(i + 1, bq)
        wait_gather(i, bp)
        start_scatter(i, bp)

    start_gather(0, 0)
    mk_step(0, 0, 1, False, True)

    def loop_body(j, carry):
        i = 2 * j + 1
        mk_step(i, 1, 0, True, True)
        mk_step(i + 1, 0, 1, True, True)
        return carry

    lax.fori_loop(0, 61, loop_body, 0)
    mk_step(123, 1, 0, True, True)
    mk_step(124, 0, 1, True, False)
    wait_scatter(124, 0)

    plsc.subcore_barrier()

    @pl.when(s < _NS - 1)
    def _():
        pltpu.sync_copy(acc.at[pl.ds(row0, _RPT_BIG)],
                        out_hbm.at[c, pl.ds(row0, _RPT_BIG)])

    @pl.when(s == _NS - 1)
    def _():
        pltpu.sync_copy(acc.at[pl.ds(row0, _RPT_LAST)],
                        out_hbm.at[c, pl.ds(row0, _RPT_LAST)])


_agg = pl.kernel(
    _agg_body,
    out_type=jax.ShapeDtypeStruct((_NC, _N, _D), jnp.float32),
    mesh=_mesh,
    scratch_types=[
        pltpu.VMEM((_EPT,), jnp.int32),
        pltpu.VMEM((_T, _K), jnp.int32),
        pltpu.VMEM((_K, _D), jnp.float32),
        pltpu.VMEM((_K, _D), jnp.float32),
        pltpu.VMEM_SHARED((_N, _D), jnp.float32),
        pltpu.SemaphoreType.DMA,
        pltpu.SemaphoreType.DMA,
        pltpu.SemaphoreType.DMA,
        pltpu.SemaphoreType.DMA,
        pltpu.SemaphoreType.DMA,
    ],
)


def _pool_body(h_hbm, batch_hbm, zero_hbm, out_hbm, bidx, rows, acc):
    c = lax.axis_index("c")
    s = lax.axis_index("s")
    w = c * _NS + s

    @pl.when(s == 0)
    def _():
        pltpu.sync_copy(zero_hbm.at[pl.ds(0, _G)], acc)

    plsc.subcore_barrier()

    nchunks = _N // _K

    def body(j, carry):
        chunk = w + j * (_NC * _NS)

        @pl.when(chunk < nchunks)
        def _():
            off = chunk * _K
            pltpu.sync_copy(batch_hbm.at[pl.ds(off, _K)], bidx)
            pltpu.sync_copy(h_hbm.at[pl.ds(off, _K)], rows)
            pltpu.sync_copy(rows, acc.at[bidx], add=True)

        return carry

    lax.fori_loop(0, (nchunks + _NC * _NS - 1) // (_NC * _NS), body, 0)
    plsc.subcore_barrier()

    @pl.when(s == 0)
    def _():
        pltpu.sync_copy(acc, out_hbm.at[c])


_pool = pl.kernel(
    _pool_body,
    out_type=jax.ShapeDtypeStruct((_NC, _G, _D), jnp.float32),
    mesh=_mesh,
    scratch_types=[
        pltpu.VMEM((_K,), jnp.int32),
        pltpu.VMEM((_K, _D), jnp.float32),
        pltpu.VMEM_SHARED((_G, _D), jnp.float32),
    ],
)

_BLK = 400


def _mlp_body(a_ref, w1_ref, b1_ref, w2_ref, b2_ref, out_ref):
    s = a_ref[0] + a_ref[1]
    h = jnp.dot(s, w1_ref[...], preferred_element_type=jnp.float32,
                precision=lax.Precision.HIGHEST)
    h = jnp.maximum(h + b1_ref[...], 0.0)
    o = jnp.dot(h, w2_ref[...], preferred_element_type=jnp.float32,
                precision=lax.Precision.HIGHEST)
    out_ref[...] = jnp.maximum(o + b2_ref[...], 0.0)


def _mlp(a, w1, b1, w2, b2):
    return pl.pallas_call(
        _mlp_body,
        grid=(_N // _BLK,),
        in_specs=[
            pl.BlockSpec((_NC, _BLK, _D), lambda i: (0, i, 0)),
            pl.BlockSpec((_D, _D), lambda i: (0, 0)),
            pl.BlockSpec((1, _D), lambda i: (0, 0)),
            pl.BlockSpec((_D, _D), lambda i: (0, 0)),
            pl.BlockSpec((1, _D), lambda i: (0, 0)),
        ],
        out_specs=pl.BlockSpec((_BLK, _D), lambda i: (i, 0)),
        out_shape=jax.ShapeDtypeStruct((_N, _D), jnp.float32),
    )(a, w1, b1, w2, b2)


def _final_body(p_ref, w_ref, b_ref, out_ref):
    pooled = p_ref[0] + p_ref[1]
    z = jnp.dot(pooled, w_ref[...], preferred_element_type=jnp.float32,
                precision=lax.Precision.HIGHEST)
    out_ref[...] = z + b_ref[...]


def _final(p, lin_W, lin_b):
    return pl.pallas_call(
        _final_body,
        out_shape=jax.ShapeDtypeStruct((_G, 1), jnp.float32),
    )(p, lin_W, lin_b.reshape(1, 1))


def kernel(x, edge_index, batch,
           c1_W1, c1_b1, c1_g, c1_be, c1_W2, c1_b2,
           c2_W1, c2_b1, c2_g, c2_be, c2_W2, c2_b2,
           c3_W1, c3_b1, c3_g, c3_be, c3_W2, c3_b2,
           lin_W, lin_b):
    src = edge_index[0].reshape(_NC, _NS, _EPT)
    dst = edge_index[1].reshape(_NC, _NS, _T, _K)
    zeros = jnp.zeros((_N, _D), jnp.float32)

    h = x
    for (W1, b1, g, be, W2, b2) in (
            (c1_W1, c1_b1, c1_g, c1_be, c1_W2, c1_b2),
            (c2_W1, c2_b1, c2_g, c2_be, c2_W2, c2_b2),
            (c3_W1, c3_b1, c3_g, c3_be, c3_W2, c3_b2)):
        scale = g / jnp.sqrt(1.0 + _BN_EPS)
        w1f = W1 * scale[None, :]
        b1f = (b1 * scale + be)[None, :]
        a = _agg(h, src, dst, zeros)
        h = _mlp(a, w1f, b1f, W2, b2[None, :])

    p = _pool(h, batch, zeros)
    return _final(p, lin_W, lin_b)


# speedup vs baseline: 11.9132x; 1.1369x over previous
import jax
import jax.numpy as jnp
import numpy as np
from jax import lax
from jax.experimental import pallas as pl
from jax.experimental.pallas import tpu as pltpu
from jax.experimental.pallas import tpu_sc as plsc

_N = 10000
_E = 320000
_D = 128
_G = 64
_BN_EPS = 1e-5

_NC = 2
_NS = 16
_K = 80
_EPT = _E // (_NC * _NS)
_T = _EPT // _K
_RPT_BIG = 632
_RPT_LAST = _N - 15 * _RPT_BIG

_mesh = plsc.VectorSubcoreMesh(core_axis_name="c", subcore_axis_name="s")


def _agg_body(x_hbm, src_hbm, dst_hbm, zero_hbm, out_hbm,
              fr0, fr1, fr2, fr3,
              is0, is1, is2, is3, is4, is5,
              id0, id1, id2, id3, id4, id5, acc,
              si0, si1, si2, si3, si4, si5,
              sg0, sg1, sg2, sg3, ss0, ss1, ss2, ss3):
    c = lax.axis_index("c")
    s = lax.axis_index("s")

    row0 = s * _RPT_BIG

    def init_from(src_hbm_ref):
        @pl.when(s < _NS - 1)
        def _():
            pltpu.sync_copy(src_hbm_ref.at[pl.ds(row0, _RPT_BIG)],
                            acc.at[pl.ds(row0, _RPT_BIG)])

        @pl.when(s == _NS - 1)
        def _():
            pltpu.sync_copy(src_hbm_ref.at[pl.ds(row0, _RPT_LAST)],
                            acc.at[pl.ds(row0, _RPT_LAST)])

    @pl.when(c == 0)
    def _():
        init_from(x_hbm)

    @pl.when(c != 0)
    def _():
        init_from(zero_hbm)

    plsc.subcore_barrier()

    ebase = (c * _NS + s) * _EPT
    fr = (fr0, fr1, fr2, fr3)
    idx_s = (is0, is1, is2, is3, is4, is5)
    idx_d = (id0, id1, id2, id3, id4, id5)
    sem_i = (si0, si1, si2, si3, si4, si5)
    sem_g = (sg0, sg1, sg2, sg3)
    sem_s = (ss0, ss1, ss2, ss3)

    def start_idx(i, b):
        off = ebase + i * _K
        pltpu.async_copy(src_hbm.at[pl.ds(off, _K)], idx_s[b], sem_i[b])
        pltpu.async_copy(dst_hbm.at[pl.ds(off, _K)], idx_d[b], sem_i[b])

    def wait_idx(b):
        pltpu.make_async_copy(src_hbm.at[pl.ds(0, _K)], idx_s[b],
                              sem_i[b]).wait()
        pltpu.make_async_copy(dst_hbm.at[pl.ds(0, _K)], idx_d[b],
                              sem_i[b]).wait()

    def start_gather(b3, b4):
        pltpu.async_copy(x_hbm.at[idx_s[b4]], fr[b3], sem_g[b3])

    def wait_gather(b3, b4):
        pltpu.make_async_copy(x_hbm.at[idx_s[b4]], fr[b3],
                              sem_g[b3]).wait()

    def start_scatter(b3, b4):
        pltpu.async_copy(fr[b3], acc.at[idx_d[b4]], sem_s[b3], add=True)

    def wait_scatter(b3, b4):
        pltpu.make_async_copy(fr[b3], acc.at[idx_d[b4]],
                              sem_s[b3]).wait()

    def pstep(i, m4, m6, with_idx=True, with_next=True, with_scwait=True):
        wait_gather(m4, m6)
        if with_scwait:
            wait_scatter((m4 + 2) % 4, (m6 + 4) % 6)
        if with_idx:
            start_idx(i + 3, (m6 + 3) % 6)
        if with_next:
            wait_idx((m6 + 2) % 6)
            start_gather((m4 + 2) % 4, (m6 + 2) % 6)
        start_scatter(m4, m6)

    start_idx(0, 0)
    start_idx(1, 1)
    start_idx(2, 2)
    wait_idx(0)
    start_gather(0, 0)
    wait_idx(1)
    start_gather(1, 1)
    pstep(0, 0, 0, with_scwait=False)
    pstep(1, 1, 1, with_scwait=False)

    def loop_body(j, carry):
        i = 12 * j + 2
        for t in range(12):
            pstep(i + t, (2 + t) % 4, (2 + t) % 6)
        return carry

    lax.fori_loop(0, 10, loop_body, 0)
    pstep(122, 2, 2, with_idx=False)
    pstep(123, 3, 3, with_idx=False, with_next=False)
    pstep(124, 0, 4, with_idx=False, with_next=False)
    wait_scatter(3, 3)
    wait_scatter(0, 4)

    plsc.subcore_barrier()

    @pl.when(s < _NS - 1)
    def _():
        pltpu.sync_copy(acc.at[pl.ds(row0, _RPT_BIG)],
                        out_hbm.at[c, pl.ds(row0, _RPT_BIG)])

    @pl.when(s == _NS - 1)
    def _():
        pltpu.sync_copy(acc.at[pl.ds(row0, _RPT_LAST)],
                        out_hbm.at[c, pl.ds(row0, _RPT_LAST)])


_agg = pl.kernel(
    _agg_body,
    out_type=jax.ShapeDtypeStruct((_NC, _N, _D), jnp.float32),
    mesh=_mesh,
    scratch_types=(
        [pltpu.VMEM((_K, _D), jnp.float32)] * 4
        + [pltpu.VMEM((_K,), jnp.int32)] * 12
        + [pltpu.SemaphoreType.DMA] * 0
        + [pltpu.VMEM_SHARED((_N, _D), jnp.float32)]
        + [pltpu.SemaphoreType.DMA] * 14
    ),
)


def _pool_body(h_hbm, batch_hbm, zero_hbm, out_hbm, bidx, rows, acc):
    c = lax.axis_index("c")
    s = lax.axis_index("s")
    w = c * _NS + s

    @pl.when(s == 0)
    def _():
        pltpu.sync_copy(zero_hbm.at[pl.ds(0, _G)], acc)

    plsc.subcore_barrier()

    nchunks = _N // _K

    def body(j, carry):
        chunk = w + j * (_NC * _NS)

        @pl.when(chunk < nchunks)
        def _():
            off = chunk * _K
            pltpu.sync_copy(batch_hbm.at[pl.ds(off, _K)], bidx)
            pltpu.sync_copy(h_hbm.at[pl.ds(off, _K)], rows)
            pltpu.sync_copy(rows, acc.at[bidx], add=True)

        return carry

    lax.fori_loop(0, (nchunks + _NC * _NS - 1) // (_NC * _NS), body, 0)
    plsc.subcore_barrier()

    @pl.when(s == 0)
    def _():
        pltpu.sync_copy(acc, out_hbm.at[c])


_pool = pl.kernel(
    _pool_body,
    out_type=jax.ShapeDtypeStruct((_NC, _G, _D), jnp.float32),
    mesh=_mesh,
    scratch_types=[
        pltpu.VMEM((_K,), jnp.int32),
        pltpu.VMEM((_K, _D), jnp.float32),
        pltpu.VMEM_SHARED((_G, _D), jnp.float32),
    ],
)

_BLK = 400


def _mlp_body(a_ref, w1_ref, b1_ref, w2_ref, b2_ref, out_ref):
    s = a_ref[0] + a_ref[1]
    h = jnp.dot(s, w1_ref[...], preferred_element_type=jnp.float32,
                precision=lax.Precision.HIGHEST)
    h = jnp.maximum(h + b1_ref[...], 0.0)
    o = jnp.dot(h, w2_ref[...], preferred_element_type=jnp.float32,
                precision=lax.Precision.HIGHEST)
    out_ref[...] = jnp.maximum(o + b2_ref[...], 0.0)


def _mlp(a, w1, b1, w2, b2):
    wspec = pl.BlockSpec((_D, _D), lambda i: (0, 0))
    bspec = pl.BlockSpec((1, _D), lambda i: (0, 0))
    return pl.pallas_call(
        _mlp_body,
        grid=(_N // _BLK,),
        in_specs=[
            pl.BlockSpec((_NC, _BLK, _D), lambda i: (0, i, 0)),
            wspec, bspec, wspec, bspec,
        ],
        out_specs=pl.BlockSpec((_BLK, _D), lambda i: (i, 0)),
        out_shape=jax.ShapeDtypeStruct((_N, _D), jnp.float32),
    )(a, w1, b1, w2, b2)


def _final_body(p_ref, w_ref, b_ref, out_ref):
    pooled = p_ref[0] + p_ref[1]
    z = jnp.dot(pooled, w_ref[...], preferred_element_type=jnp.float32,
                precision=lax.Precision.HIGHEST)
    out_ref[...] = z + b_ref[...]


def _final(p, lin_W, lin_b):
    return pl.pallas_call(
        _final_body,
        out_shape=jax.ShapeDtypeStruct((_G, 1), jnp.float32),
    )(p, lin_W, lin_b.reshape(1, 1))


def kernel(x, edge_index, batch,
           c1_W1, c1_b1, c1_g, c1_be, c1_W2, c1_b2,
           c2_W1, c2_b1, c2_g, c2_be, c2_W2, c2_b2,
           c3_W1, c3_b1, c3_g, c3_be, c3_W2, c3_b2,
           lin_W, lin_b):
    src = edge_index[0]
    dst = edge_index[1]
    zeros = jnp.zeros((_N, _D), jnp.float32)

    h = x
    for (W1, b1, g, be, W2, b2) in (
            (c1_W1, c1_b1, c1_g, c1_be, c1_W2, c1_b2),
            (c2_W1, c2_b1, c2_g, c2_be, c2_W2, c2_b2),
            (c3_W1, c3_b1, c3_g, c3_be, c3_W2, c3_b2)):
        scale = g / jnp.sqrt(1.0 + _BN_EPS)
        w1f = W1 * scale[None, :]
        b1f = (b1 * scale + be)[None, :]
        a = _agg(h, src, dst, zeros)
        h = _mlp(a, w1f, b1f, W2, b2[None, :])

    p = _pool(h, batch, zeros)
    return _final(p, lin_W, lin_b)
